# Initial kernel scaffold; baseline (speedup 1.0000x reference)
import jax
import jax.numpy as jnp
from jax.experimental import pallas as pl


def kernel(z, means, logvars, w):
    raise NotImplementedError("write your pallas kernel here")


# speedup vs baseline: 1.2841x; 1.2841x over previous
import functools
import math

import jax
import jax.numpy as jnp
from jax.experimental import pallas as pl
from jax.experimental.pallas import tpu as pltpu

LOG2PI = math.log(2.0 * math.pi)


def _prep_body(mu_ref, lv_ref, w_ref, a_ref, b_ref, c_ref):
    mu = mu_ref[...]
    lv = lv_ref[...]
    w = w_ref[...]
    wm = jnp.max(w)
    lw = w - (wm + jnp.log(jnp.sum(jnp.exp(w - wm))))
    ev = jnp.exp(-lv)
    a_ref[...] = -0.5 * ev
    b_ref[...] = ev * mu
    c_ref[...] = -0.5 * (LOG2PI + lv + ev * mu * mu) + lw


def _main_body(z_ref, a_ref, b_ref, c_ref, o_ref, *, L, K, KC):
    nchunk = K // KC
    for l in range(L):
        zl = z_ref[:, l : l + 1]
        z2l = zl * zl
        a = a_ref[l : l + 1, :]
        b = b_ref[l : l + 1, :]
        c = c_ref[l : l + 1, :]
        macc = None
        for i in range(nchunk):
            sl = slice(i * KC, (i + 1) * KC)
            t = z2l * a[:, sl] + zl * b[:, sl] + c[:, sl]
            macc = t if macc is None else jnp.maximum(macc, t)
        m_l = jnp.max(macc, axis=1, keepdims=True)
        sacc = None
        for i in range(nchunk):
            sl = slice(i * KC, (i + 1) * KC)
            t = z2l * a[:, sl] + zl * b[:, sl] + c[:, sl]
            e = jnp.exp(t - m_l)
            sacc = e if sacc is None else sacc + e
        s_l = jnp.sum(sacc, axis=1, keepdims=True)
        o_ref[:, l : l + 1] = m_l + jnp.log(s_l)


def kernel(z, means, logvars, w):
    B, L = z.shape
    K = means.shape[0]
    mu_t = means.T
    lv_t = logvars.T
    w2 = w.reshape(1, K)

    a_t, b_t, c_t = pl.pallas_call(
        _prep_body,
        out_shape=[jax.ShapeDtypeStruct((L, K), jnp.float32)] * 3,
    )(mu_t, lv_t, w2)

    BT = 128
    KC = 128
    out = pl.pallas_call(
        functools.partial(_main_body, L=L, K=K, KC=KC),
        grid=(B // BT,),
        in_specs=[
            pl.BlockSpec((BT, L), lambda i: (i, 0)),
            pl.BlockSpec((L, K), lambda i: (0, 0)),
            pl.BlockSpec((L, K), lambda i: (0, 0)),
            pl.BlockSpec((L, K), lambda i: (0, 0)),
        ],
        out_specs=pl.BlockSpec((BT, L), lambda i: (i, 0)),
        out_shape=jax.ShapeDtypeStruct((B, L), jnp.float32),
    )(z, a_t, b_t, c_t)
    return out
